# baseline (speedup 1.0000x reference)
import functools

import jax
import jax.numpy as jnp
from jax import lax
from jax.experimental import pallas as pl
from jax.experimental.pallas import tpu as pltpu
from jax.experimental.pallas import tpu_sc as plsc

N = 50000
EMB = 64
NE = 800000

NC = 2
NS = 16
NW = NC * NS
HALF = N // 2
TBL = 25088
STRIPE = TBL // NS
DUMMY = HALF
C = 128
NBUF = 3
NBUFD = 5
M = 210
CAPW = 64 * M
CHW = CAPW // C
NPART = NW * CAPW
EPP = NW * 25088
EW = EPP // NW
BTC = 2000


def _mesh():
    return plsc.VectorSubcoreMesh(
        core_axis_name="c", subcore_axis_name="s", num_cores=NC,
        num_subcores=NS)


_SC_PARAMS = pltpu.CompilerParams(use_tc_tiling_on_sc=False,
                                  internal_scratch_in_bytes=131072)
_SC_PARAMS_NL = pltpu.CompilerParams(use_tc_tiling_on_sc=False,
                                     internal_scratch_in_bytes=131072,
                                     needs_layout_passes=False)


def _sc_partition(rowp, colp):
    @functools.partial(
        pl.kernel,
        out_type=jax.ShapeDtypeStruct((NC, NW, CHW * 2 * C), jnp.int32),
        mesh=_mesh(),
        scratch_types=[
            pltpu.VMEM((EW,), jnp.int32),
            pltpu.VMEM((EW,), jnp.int32),
            pltpu.VMEM((2 * CHW * 2 * C,), jnp.int32),
            pltpu.SemaphoreType.DMA,
        ],
        compiler_params=_SC_PARAMS_NL,
    )
    def k(row_hbm, col_hbm, out_hbm, rin, cin, buf, sem):
        cid = lax.axis_index("c")
        sid = lax.axis_index("s")
        w = cid * NS + sid

        cr = pltpu.async_copy(row_hbm.at[pl.ds(w * EW, EW)], rin, sem)
        cc = pltpu.async_copy(col_hbm.at[pl.ds(w * EW, EW)], cin, sem)

        iota = lax.iota(jnp.int32, 16)
        zero16 = jnp.zeros((16,), jnp.int32)
        dummyv = DUMMY + iota

        @pl.loop(0, 2 * CHW)
        def _(j):
            dv = dummyv + (j & 3) * 16

            @pl.loop(0, C // 16)
            def _(i):
                buf[pl.ds(j * 2 * C + i * 16, 16)] = zero16
                buf[pl.ds(j * 2 * C + C + i * 16, 16)] = dv

        cr.wait()
        cc.wait()

        def body(i, carry):
            cnt0, cnt1 = carry
            colv = cin[pl.ds(i * 16, 16)]
            rowv = rin[pl.ds(i * 16, 16)]
            m = colv < HALF
            mi = m.astype(jnp.int32)
            c = plsc.cumsum(mi)
            s = jnp.sum(mi)
            r = jnp.where(m, cnt0 + c - 1, cnt1 + iota - c)
            ch = jnp.where(m, r >> 7, CHW + (r >> 7))
            ch = jnp.minimum(ch, 2 * CHW - 1)
            pos = ch * (2 * C) + (r & (C - 1))
            colloc = jnp.where(m, colv, colv - HALF)
            plsc.store_scatter(buf, [pos], rowv)
            plsc.store_scatter(buf, [pos + C], colloc)
            return (cnt0 + s, cnt1 + 16 - s)

        pl.loop(0, EW // 16, init_carry=(jnp.int32(0), jnp.int32(0)))(body)

        half_w = CHW * 2 * C
        pltpu.sync_copy(buf.at[pl.ds(0, half_w)], out_hbm.at[0, w])
        pltpu.sync_copy(buf.at[pl.ds(half_w, half_w)], out_hbm.at[1, w])

    return k(rowp, colp)


def _sc_deg(rc, ones_hbm, z_hbm):
    @functools.partial(
        pl.kernel,
        out_type=jax.ShapeDtypeStruct((NC, TBL, 16), jnp.float32),
        mesh=_mesh(),
        scratch_types=[
            pltpu.VMEM((NBUFD, C), jnp.int32),
            pltpu.VMEM((C, 16), jnp.float32),
            pltpu.VMEM_SHARED((TBL, 16), jnp.float32),
            pltpu.SemaphoreType.DMA,
            pltpu.SemaphoreType.DMA,
            pltpu.SemaphoreType.DMA,
            pltpu.SemaphoreType.DMA,
            pltpu.SemaphoreType.DMA,
        ],
        compiler_params=_SC_PARAMS,
    )
    def k(rc_hbm, ones_hbm, z_hbm, out_hbm, cidx, ones_v, table, s0, s1,
          s2, s3, s4):
        cid = lax.axis_index("c")
        sid = lax.axis_index("s")
        ssem = [s0, s1, s2, s3, s4]
        pltpu.sync_copy(ones_hbm, ones_v)
        pltpu.sync_copy(z_hbm, table.at[pl.ds(sid * STRIPE, STRIPE)])
        plsc.subcore_barrier()

        base = sid * M
        for b in range(NBUFD):
            pltpu.sync_copy(rc_hbm.at[cid, base + b, 1], cidx.at[b])
            pltpu.async_copy(ones_v, table.at[cidx.at[b]], ssem[b], add=True)

        @pl.loop(1, M // NBUFD)
        def _(it):
            for b in range(NBUFD):
                pltpu.make_async_copy(z_hbm.at[pl.ds(0, C)], ones_v,
                                      ssem[b]).wait()
                pltpu.sync_copy(rc_hbm.at[cid, base + it * NBUFD + b, 1],
                                cidx.at[b])
                pltpu.async_copy(ones_v, table.at[cidx.at[b]], ssem[b],
                                 add=True)

        for b in range(NBUFD):
            pltpu.make_async_copy(z_hbm.at[pl.ds(0, C)], ones_v,
                                  ssem[b]).wait()

        plsc.subcore_barrier()
        pltpu.sync_copy(table.at[pl.ds(sid * STRIPE, STRIPE)],
                        out_hbm.at[cid, pl.ds(sid * STRIPE, STRIPE)])

    return k(rc, ones_hbm, z_hbm)


def _sc_prop(xs, rc, z64):
    @functools.partial(
        pl.kernel,
        out_type=jax.ShapeDtypeStruct((NC, TBL, EMB), jnp.float32),
        mesh=_mesh(),
        scratch_types=[
            pltpu.VMEM((NBUF, 2, C), jnp.int32),
            pltpu.VMEM((NBUF, C, EMB), jnp.float32),
            pltpu.VMEM_SHARED((TBL, EMB), jnp.float32),
            pltpu.SemaphoreType.DMA,
            pltpu.SemaphoreType.DMA,
            pltpu.SemaphoreType.DMA,
            pltpu.SemaphoreType.DMA,
            pltpu.SemaphoreType.DMA,
            pltpu.SemaphoreType.DMA,
        ],
        compiler_params=_SC_PARAMS,
    )
    def k(xs_hbm, rc_hbm, z_hbm, out_hbm, rcidx, rows_v, table,
          g0, g1, g2, s0, s1, s2):
        cid = lax.axis_index("c")
        sid = lax.axis_index("s")
        gsem = [g0, g1, g2]
        ssem = [s0, s1, s2]
        pltpu.sync_copy(z_hbm, table.at[pl.ds(sid * STRIPE, STRIPE)])
        plsc.subcore_barrier()

        base = sid * M
        for b in range(NBUF):
            pltpu.sync_copy(rc_hbm.at[cid, base + b], rcidx.at[b])
            pltpu.async_copy(xs_hbm.at[rcidx.at[b, 0]], rows_v.at[b],
                             gsem[b])

        @pl.loop(0, M // NBUF)
        def _(it):
            for b in range(NBUF):
                ci = it * NBUF + b
                prev = (b - 1) % NBUF
                pltpu.make_async_copy(xs_hbm.at[pl.ds(0, C)], rows_v.at[b],
                                      gsem[b]).wait()
                pltpu.async_copy(rows_v.at[b], table.at[rcidx.at[b, 1]],
                                 ssem[b], add=True)

                @pl.when(jnp.logical_and(ci >= 1, ci <= M - NBUF))
                def _():
                    pltpu.make_async_copy(xs_hbm.at[pl.ds(0, C)],
                                          rows_v.at[prev],
                                          ssem[prev]).wait()
                    pltpu.sync_copy(rc_hbm.at[cid, base + ci - 1 + NBUF],
                                    rcidx.at[prev])
                    pltpu.async_copy(xs_hbm.at[rcidx.at[prev, 0]],
                                     rows_v.at[prev], gsem[prev])

        for b in range(NBUF):
            pltpu.make_async_copy(xs_hbm.at[pl.ds(0, C)], rows_v.at[b],
                                  ssem[b]).wait()

        plsc.subcore_barrier()
        pltpu.sync_copy(table.at[pl.ds(sid * STRIPE, STRIPE)],
                        out_hbm.at[cid, pl.ds(sid * STRIPE, STRIPE)])

    return k(xs, rc, z64)


def _dis_block(deg_blk):
    d = deg_blk[:, 0:1]
    return jnp.where(d > 0, lax.rsqrt(d), 0.0)


def _tc_prescale(E, deg):
    def body(deg_ref, e_ref, o_ref):
        o_ref[...] = e_ref[...] * _dis_block(deg_ref[...])

    return pl.pallas_call(
        body,
        grid=(N // BTC,),
        in_specs=[
            pl.BlockSpec((BTC, 16), lambda i: (i, 0)),
            pl.BlockSpec((BTC, EMB), lambda i: (i, 0)),
        ],
        out_specs=pl.BlockSpec((BTC, EMB), lambda i: (i, 0)),
        out_shape=jax.ShapeDtypeStruct((N, EMB), jnp.float32),
    )(deg, E)


def _tc_post(agg, deg):
    def body(agg_ref, deg_ref, x_ref, xs_ref):
        dis = _dis_block(deg_ref[...])
        t = agg_ref[...] * dis
        t = jnp.where(t >= 0, t, 0.01 * t)
        nrm = jnp.sqrt(jnp.sum(t * t, axis=1, keepdims=True))
        x = t / jnp.maximum(nrm, 1e-12)
        x_ref[...] = x
        xs_ref[...] = x * dis

    return pl.pallas_call(
        body,
        grid=(N // BTC,),
        in_specs=[
            pl.BlockSpec((BTC, EMB), lambda i: (i, 0)),
            pl.BlockSpec((BTC, 16), lambda i: (i, 0)),
        ],
        out_specs=[
            pl.BlockSpec((BTC, EMB), lambda i: (i, 0)),
            pl.BlockSpec((BTC, EMB), lambda i: (i, 0)),
        ],
        out_shape=[
            jax.ShapeDtypeStruct((N, EMB), jnp.float32),
            jax.ShapeDtypeStruct((N, EMB), jnp.float32),
        ],
    )(agg, deg)


def _tc_final(E, x1, x2, x3):
    def body(e_ref, a_ref, b_ref, c_ref, o_ref):
        o_ref[...] = 0.25 * (e_ref[...] + a_ref[...] + b_ref[...]
                             + c_ref[...])

    spec = pl.BlockSpec((BTC, EMB), lambda i: (i, 0))
    return pl.pallas_call(
        body,
        grid=(N // BTC,),
        in_specs=[spec, spec, spec, spec],
        out_specs=spec,
        out_shape=jax.ShapeDtypeStruct((N, EMB), jnp.float32),
    )(E, x1, x2, x3)


def kernel(edge_index, E):
    row = edge_index[0]
    col = edge_index[1]
    pad = EPP - NE

    e = jnp.arange(pad, dtype=jnp.int32)
    rowp = jnp.concatenate([row, jnp.zeros((pad,), jnp.int32)])
    colp = jnp.concatenate([col, HALF + DUMMY + (e & 15)])

    rc = _sc_partition(rowp, colp)
    rc = rc.reshape(NC, NW * CHW, 2, C)

    ones16 = jnp.ones((C, 16), jnp.float32)
    z16 = jnp.zeros((STRIPE, 16), jnp.float32)
    z64 = jnp.zeros((STRIPE, EMB), jnp.float32)

    degp = _sc_deg(rc, ones16, z16)
    deg = jnp.concatenate([degp[0, :HALF], degp[1, :HALF]], axis=0)

    xs = _tc_prescale(E, deg)
    xlist = []
    for _ in range(3):
        aggp = _sc_prop(xs, rc, z64)
        agg = jnp.concatenate([aggp[0, :HALF], aggp[1, :HALF]], axis=0)
        x, xs = _tc_post(agg, deg)
        xlist.append(x)

    return _tc_final(E, *xlist)


# speedup vs baseline: 3.6740x; 3.6740x over previous
import functools

import jax
import jax.numpy as jnp
from jax import lax
from jax.experimental import pallas as pl
from jax.experimental.pallas import tpu as pltpu
from jax.experimental.pallas import tpu_sc as plsc

N = 50000
EMB = 64
NE = 800000

NC = 2
NS = 16
NW = NC * NS
HALF = N // 2
TBL = 25088
STRIPE = TBL // NS
DUMMY = HALF
C = 128
NBUF = 3
NBUFD = 5
M = 210
CAPW = 64 * M
CHW = CAPW // C
NPART = NW * CAPW
EPP = NW * 25088
EW = EPP // NW
BTC = 2000


def _mesh():
    return plsc.VectorSubcoreMesh(
        core_axis_name="c", subcore_axis_name="s", num_cores=NC,
        num_subcores=NS)


_SC_PARAMS = pltpu.CompilerParams(use_tc_tiling_on_sc=False,
                                  internal_scratch_in_bytes=131072)
_SC_PARAMS_NL = pltpu.CompilerParams(use_tc_tiling_on_sc=False,
                                     internal_scratch_in_bytes=131072,
                                     needs_layout_passes=False)


def _sc_partition(rowp, colp):
    @functools.partial(
        pl.kernel,
        out_type=jax.ShapeDtypeStruct((NC, NW, CHW * 2 * C), jnp.int32),
        mesh=_mesh(),
        scratch_types=[
            pltpu.VMEM((EW,), jnp.int32),
            pltpu.VMEM((EW,), jnp.int32),
            pltpu.VMEM((2 * CHW * 2 * C,), jnp.int32),
            pltpu.SemaphoreType.DMA,
        ],
        compiler_params=_SC_PARAMS_NL,
    )
    def k(row_hbm, col_hbm, out_hbm, rin, cin, buf, sem):
        cid = lax.axis_index("c")
        sid = lax.axis_index("s")
        w = cid * NS + sid

        cr = pltpu.async_copy(row_hbm.at[pl.ds(w * EW, EW)], rin, sem)
        cc = pltpu.async_copy(col_hbm.at[pl.ds(w * EW, EW)], cin, sem)

        iota = lax.iota(jnp.int32, 16)
        zero16 = jnp.zeros((16,), jnp.int32)
        dummyv = DUMMY + iota

        @pl.loop(0, 2 * CHW)
        def _(j):
            dv = dummyv + (j & 3) * 16
            rv0 = iota + ((w * 2 * CHW + j) * 61) % 3000

            @pl.loop(0, C // 16)
            def _(i):
                buf[pl.ds(j * 2 * C + i * 16, 16)] = rv0 + i * 16
                buf[pl.ds(j * 2 * C + C + i * 16, 16)] = dv

        cr.wait()
        cc.wait()

        def body(i, carry):
            cnt0, cnt1 = carry
            colv = cin[pl.ds(i * 16, 16)]
            rowv = rin[pl.ds(i * 16, 16)]
            m = colv < HALF
            mi = m.astype(jnp.int32)
            c = plsc.cumsum(mi)
            s = jnp.sum(mi)
            r = jnp.where(m, cnt0 + c - 1, cnt1 + iota - c)
            ch = jnp.where(m, r >> 7, CHW + (r >> 7))
            ch = jnp.minimum(ch, 2 * CHW - 1)
            pos = ch * (2 * C) + (r & (C - 1))
            colloc = jnp.where(m, colv, colv - HALF)
            plsc.store_scatter(buf, [pos], rowv)
            plsc.store_scatter(buf, [pos + C], colloc)
            return (cnt0 + s, cnt1 + 16 - s)

        pl.loop(0, EW // 16, init_carry=(jnp.int32(0), jnp.int32(0)))(body)

        half_w = CHW * 2 * C
        pltpu.sync_copy(buf.at[pl.ds(0, half_w)], out_hbm.at[0, w])
        pltpu.sync_copy(buf.at[pl.ds(half_w, half_w)], out_hbm.at[1, w])

    return k(rowp, colp)


def _sc_deg(rc, ones_hbm, z_hbm):
    @functools.partial(
        pl.kernel,
        out_type=jax.ShapeDtypeStruct((NC, TBL, 16), jnp.float32),
        mesh=_mesh(),
        scratch_types=[
            pltpu.VMEM((NBUFD, C), jnp.int32),
            pltpu.VMEM((C, 16), jnp.float32),
            pltpu.VMEM_SHARED((TBL, 16), jnp.float32),
            pltpu.SemaphoreType.DMA,
            pltpu.SemaphoreType.DMA,
            pltpu.SemaphoreType.DMA,
            pltpu.SemaphoreType.DMA,
            pltpu.SemaphoreType.DMA,
        ],
        compiler_params=_SC_PARAMS,
    )
    def k(rc_hbm, ones_hbm, z_hbm, out_hbm, cidx, ones_v, table, s0, s1,
          s2, s3, s4):
        cid = lax.axis_index("c")
        sid = lax.axis_index("s")
        ssem = [s0, s1, s2, s3, s4]
        pltpu.sync_copy(ones_hbm, ones_v)
        pltpu.sync_copy(z_hbm, table.at[pl.ds(sid * STRIPE, STRIPE)])
        plsc.subcore_barrier()

        base = sid * M
        for b in range(NBUFD):
            pltpu.sync_copy(rc_hbm.at[cid, base + b, 1], cidx.at[b])
            pltpu.async_copy(ones_v, table.at[cidx.at[b]], ssem[b], add=True)

        @pl.loop(1, M // NBUFD)
        def _(it):
            for b in range(NBUFD):
                pltpu.make_async_copy(z_hbm.at[pl.ds(0, C)], ones_v,
                                      ssem[b]).wait()
                pltpu.sync_copy(rc_hbm.at[cid, base + it * NBUFD + b, 1],
                                cidx.at[b])
                pltpu.async_copy(ones_v, table.at[cidx.at[b]], ssem[b],
                                 add=True)

        for b in range(NBUFD):
            pltpu.make_async_copy(z_hbm.at[pl.ds(0, C)], ones_v,
                                  ssem[b]).wait()

        plsc.subcore_barrier()
        pltpu.sync_copy(table.at[pl.ds(sid * STRIPE, STRIPE)],
                        out_hbm.at[cid, pl.ds(sid * STRIPE, STRIPE)])

    return k(rc, ones_hbm, z_hbm)


def _sc_prop(xs, rc, z64):
    @functools.partial(
        pl.kernel,
        out_type=jax.ShapeDtypeStruct((NC, TBL, EMB), jnp.float32),
        mesh=_mesh(),
        scratch_types=[
            pltpu.VMEM((NBUF, 2, C), jnp.int32),
            pltpu.VMEM((NBUF, C, EMB), jnp.float32),
            pltpu.VMEM_SHARED((TBL, EMB), jnp.float32),
            pltpu.SemaphoreType.DMA,
            pltpu.SemaphoreType.DMA,
            pltpu.SemaphoreType.DMA,
            pltpu.SemaphoreType.DMA,
            pltpu.SemaphoreType.DMA,
            pltpu.SemaphoreType.DMA,
        ],
        compiler_params=_SC_PARAMS,
    )
    def k(xs_hbm, rc_hbm, z_hbm, out_hbm, rcidx, rows_v, table,
          g0, g1, g2, s0, s1, s2):
        cid = lax.axis_index("c")
        sid = lax.axis_index("s")
        gsem = [g0, g1, g2]
        ssem = [s0, s1, s2]
        pltpu.sync_copy(z_hbm, table.at[pl.ds(sid * STRIPE, STRIPE)])
        plsc.subcore_barrier()

        base = sid * M
        for b in range(NBUF):
            pltpu.sync_copy(rc_hbm.at[cid, base + b], rcidx.at[b])
            pltpu.async_copy(xs_hbm.at[rcidx.at[b, 0]], rows_v.at[b],
                             gsem[b])

        @pl.loop(0, M // NBUF)
        def _(it):
            for b in range(NBUF):
                ci = it * NBUF + b
                prev = (b - 1) % NBUF
                pltpu.make_async_copy(xs_hbm.at[pl.ds(0, C)], rows_v.at[b],
                                      gsem[b]).wait()
                pltpu.async_copy(rows_v.at[b], table.at[rcidx.at[b, 1]],
                                 ssem[b], add=True)

                @pl.when(jnp.logical_and(ci >= 1, ci <= M - NBUF))
                def _():
                    pltpu.make_async_copy(xs_hbm.at[pl.ds(0, C)],
                                          rows_v.at[prev],
                                          ssem[prev]).wait()
                    pltpu.sync_copy(rc_hbm.at[cid, base + ci - 1 + NBUF],
                                    rcidx.at[prev])
                    pltpu.async_copy(xs_hbm.at[rcidx.at[prev, 0]],
                                     rows_v.at[prev], gsem[prev])

        for b in range(NBUF):
            pltpu.make_async_copy(xs_hbm.at[pl.ds(0, C)], rows_v.at[b],
                                  ssem[b]).wait()

        plsc.subcore_barrier()
        pltpu.sync_copy(table.at[pl.ds(sid * STRIPE, STRIPE)],
                        out_hbm.at[cid, pl.ds(sid * STRIPE, STRIPE)])

    return k(xs, rc, z64)


def _dis_block(deg_blk):
    d = deg_blk[:, 0:1]
    return jnp.where(d > 0, lax.rsqrt(d), 0.0)


def _tc_prescale(E, deg):
    def body(deg_ref, e_ref, o_ref):
        o_ref[...] = e_ref[...] * _dis_block(deg_ref[...])

    return pl.pallas_call(
        body,
        grid=(N // BTC,),
        in_specs=[
            pl.BlockSpec((BTC, 16), lambda i: (i, 0)),
            pl.BlockSpec((BTC, EMB), lambda i: (i, 0)),
        ],
        out_specs=pl.BlockSpec((BTC, EMB), lambda i: (i, 0)),
        out_shape=jax.ShapeDtypeStruct((N, EMB), jnp.float32),
    )(deg, E)


def _tc_post(agg, deg):
    def body(agg_ref, deg_ref, x_ref, xs_ref):
        dis = _dis_block(deg_ref[...])
        t = agg_ref[...] * dis
        t = jnp.where(t >= 0, t, 0.01 * t)
        nrm = jnp.sqrt(jnp.sum(t * t, axis=1, keepdims=True))
        x = t / jnp.maximum(nrm, 1e-12)
        x_ref[...] = x
        xs_ref[...] = x * dis

    return pl.pallas_call(
        body,
        grid=(N // BTC,),
        in_specs=[
            pl.BlockSpec((BTC, EMB), lambda i: (i, 0)),
            pl.BlockSpec((BTC, 16), lambda i: (i, 0)),
        ],
        out_specs=[
            pl.BlockSpec((BTC, EMB), lambda i: (i, 0)),
            pl.BlockSpec((BTC, EMB), lambda i: (i, 0)),
        ],
        out_shape=[
            jax.ShapeDtypeStruct((N, EMB), jnp.float32),
            jax.ShapeDtypeStruct((N, EMB), jnp.float32),
        ],
    )(agg, deg)


def _tc_final(E, x1, x2, x3):
    def body(e_ref, a_ref, b_ref, c_ref, o_ref):
        o_ref[...] = 0.25 * (e_ref[...] + a_ref[...] + b_ref[...]
                             + c_ref[...])

    spec = pl.BlockSpec((BTC, EMB), lambda i: (i, 0))
    return pl.pallas_call(
        body,
        grid=(N // BTC,),
        in_specs=[spec, spec, spec, spec],
        out_specs=spec,
        out_shape=jax.ShapeDtypeStruct((N, EMB), jnp.float32),
    )(E, x1, x2, x3)


def kernel(edge_index, E):
    row = edge_index[0]
    col = edge_index[1]
    pad = EPP - NE

    e = jnp.arange(pad, dtype=jnp.int32)
    rowp = jnp.concatenate([row, e & 4095])
    colp = jnp.concatenate([col, HALF + DUMMY + (e & 15)])

    rc = _sc_partition(rowp, colp)
    rc = rc.reshape(NC, NW * CHW, 2, C)

    ones16 = jnp.ones((C, 16), jnp.float32)
    z16 = jnp.zeros((STRIPE, 16), jnp.float32)
    z64 = jnp.zeros((STRIPE, EMB), jnp.float32)

    degp = _sc_deg(rc, ones16, z16)
    deg = jnp.concatenate([degp[0, :HALF], degp[1, :HALF]], axis=0)

    xs = _tc_prescale(E, deg)
    xlist = []
    for _ in range(3):
        aggp = _sc_prop(xs, rc, z64)
        agg = jnp.concatenate([aggp[0, :HALF], aggp[1, :HALF]], axis=0)
        x, xs = _tc_post(agg, deg)
        xlist.append(x)

    return _tc_final(E, *xlist)
